# baseline (speedup 1.0000x reference)
import jax
import jax.numpy as jnp
from jax import lax
from jax.experimental import pallas as pl
from jax.experimental.pallas import tpu as pltpu
from jax.experimental.pallas import tpu_sc as plsc

B = 16384
D = 32
V = 1000000
L = 16
NC = 2
NS = 16
NW = NC * NS
BPW = B // NW
CHUNK = 128
NCHUNK = BPW // CHUNK

BLK = 16384
GRID = (V + BLK - 1) // BLK


def _rowsum_body(ut_ref, it_ref, ru_ref, ri_ref):
    o = jnp.ones((8, D), jnp.float32)
    ru_ref[...] = jnp.dot(o, ut_ref[...],
                          preferred_element_type=jnp.float32)[0]
    ri_ref[...] = jnp.dot(o, it_ref[...],
                          preferred_element_type=jnp.float32)[0]


def _tc_rowsums(ut_t, it_t):
    return pl.pallas_call(
        _rowsum_body,
        grid=(GRID,),
        in_specs=[
            pl.BlockSpec((D, BLK), lambda i: (0, i)),
            pl.BlockSpec((D, BLK), lambda i: (0, i)),
        ],
        out_specs=[
            pl.BlockSpec((BLK,), lambda i: (i,)),
            pl.BlockSpec((BLK,), lambda i: (i,)),
        ],
        out_shape=[
            jax.ShapeDtypeStruct((V,), jnp.float32),
            jax.ShapeDtypeStruct((V,), jnp.float32),
        ],
    )(ut_t, it_t)


def _gather_body(ub_hbm, ib_hbm, ru_hbm, ri_hbm, out_hbm,
                 uidx, iidx, gu, gi, outv, sem):
    wid = lax.axis_index("s") * NC + lax.axis_index("c")
    base = wid * BPW

    for j in range(NCHUNK):
        pltpu.sync_copy(ub_hbm.at[pl.ds(base + j * CHUNK, CHUNK)], uidx.at[j])
        pltpu.sync_copy(ib_hbm.at[pl.ds(base + j * CHUNK, CHUNK)], iidx.at[j])

    copies = []
    for j in range(NCHUNK):
        copies.append(pltpu.async_copy(
            ru_hbm.at[uidx.at[j]], gu.at[j], sem))
        copies.append(pltpu.async_copy(
            ri_hbm.at[iidx.at[j]], gi.at[j], sem))
    for c in copies:
        c.wait()

    for j in range(NCHUNK):
        for k in range(CHUNK // L):
            cs = k * L
            s = gu[j, pl.ds(cs, L)] * gi[j, pl.ds(cs, L)]
            outv[pl.ds(j * CHUNK + cs, L)] = 1.0 / (1.0 + jnp.exp(-s))

    pltpu.sync_copy(outv, out_hbm.at[pl.ds(base, BPW)])


def _sc_gather(user_batch, item_batch, rs_u, rs_i):
    mesh = plsc.VectorSubcoreMesh(core_axis_name="c", subcore_axis_name="s")
    run = pl.kernel(
        _gather_body,
        out_type=jax.ShapeDtypeStruct((B,), jnp.float32),
        mesh=mesh,
        scratch_types=[
            pltpu.VMEM((NCHUNK, CHUNK), jnp.int32),
            pltpu.VMEM((NCHUNK, CHUNK), jnp.int32),
            pltpu.VMEM((NCHUNK, CHUNK), jnp.float32),
            pltpu.VMEM((NCHUNK, CHUNK), jnp.float32),
            pltpu.VMEM((BPW,), jnp.float32),
            pltpu.SemaphoreType.DMA,
        ],
        compiler_params=pltpu.CompilerParams(
            needs_layout_passes=False, use_tc_tiling_on_sc=False),
    )
    return run(user_batch, item_batch, rs_u, rs_i)


def kernel(user_batch, item_batch, user_table, item_table):
    rs_u, rs_i = _tc_rowsums(user_table.T, item_table.T)
    return _sc_gather(user_batch.astype(jnp.int32),
                      item_batch.astype(jnp.int32), rs_u, rs_i)


# speedup vs baseline: 1.0805x; 1.0805x over previous
import jax
import jax.numpy as jnp
from jax import lax
from jax.experimental import pallas as pl
from jax.experimental.pallas import tpu as pltpu
from jax.experimental.pallas import tpu_sc as plsc

B = 16384
D = 32
V = 1000000
L = 16
NC = 2
NS = 16
NW = NC * NS
BPW = B // NW
CHUNK = 128
NCHUNK = BPW // CHUNK

BLK = 32768
GRID = (V + BLK - 1) // BLK


def _rowsum_body(ut_ref, it_ref, ru_ref, ri_ref):
    o = jnp.ones((8, D), jnp.float32)
    ru_ref[...] = jnp.dot(o, ut_ref[...],
                          preferred_element_type=jnp.float32)[0]
    ri_ref[...] = jnp.dot(o, it_ref[...],
                          preferred_element_type=jnp.float32)[0]


def _tc_rowsums(ut_t, it_t):
    return pl.pallas_call(
        _rowsum_body,
        grid=(GRID,),
        in_specs=[
            pl.BlockSpec((D, BLK), lambda i: (0, i)),
            pl.BlockSpec((D, BLK), lambda i: (0, i)),
        ],
        out_specs=[
            pl.BlockSpec((BLK,), lambda i: (i,)),
            pl.BlockSpec((BLK,), lambda i: (i,)),
        ],
        out_shape=[
            jax.ShapeDtypeStruct((V,), jnp.float32),
            jax.ShapeDtypeStruct((V,), jnp.float32),
        ],
    )(ut_t, it_t)


def _gather_body(ub_hbm, ib_hbm, ru_hbm, ri_hbm, out_hbm,
                 uidx, iidx, gu, gi, outv, sem):
    wid = lax.axis_index("s") * NC + lax.axis_index("c")
    base = wid * BPW

    for j in range(NCHUNK):
        pltpu.sync_copy(ub_hbm.at[pl.ds(base + j * CHUNK, CHUNK)], uidx.at[j])
        pltpu.sync_copy(ib_hbm.at[pl.ds(base + j * CHUNK, CHUNK)], iidx.at[j])

    copies = []
    for j in range(NCHUNK):
        copies.append(pltpu.async_copy(
            ru_hbm.at[uidx.at[j]], gu.at[j], sem))
        copies.append(pltpu.async_copy(
            ri_hbm.at[iidx.at[j]], gi.at[j], sem))
    for c in copies:
        c.wait()

    for j in range(NCHUNK):
        for k in range(CHUNK // L):
            cs = k * L
            s = gu[j, pl.ds(cs, L)] * gi[j, pl.ds(cs, L)]
            outv[pl.ds(j * CHUNK + cs, L)] = 1.0 / (1.0 + jnp.exp(-s))

    pltpu.sync_copy(outv, out_hbm.at[pl.ds(base, BPW)])


def _sc_gather(user_batch, item_batch, rs_u, rs_i):
    mesh = plsc.VectorSubcoreMesh(core_axis_name="c", subcore_axis_name="s")
    run = pl.kernel(
        _gather_body,
        out_type=jax.ShapeDtypeStruct((B,), jnp.float32),
        mesh=mesh,
        scratch_types=[
            pltpu.VMEM((NCHUNK, CHUNK), jnp.int32),
            pltpu.VMEM((NCHUNK, CHUNK), jnp.int32),
            pltpu.VMEM((NCHUNK, CHUNK), jnp.float32),
            pltpu.VMEM((NCHUNK, CHUNK), jnp.float32),
            pltpu.VMEM((BPW,), jnp.float32),
            pltpu.SemaphoreType.DMA,
        ],
        compiler_params=pltpu.CompilerParams(
            needs_layout_passes=False, use_tc_tiling_on_sc=False),
    )
    return run(user_batch, item_batch, rs_u, rs_i)


def kernel(user_batch, item_batch, user_table, item_table):
    rs_u, rs_i = _tc_rowsums(user_table.T, item_table.T)
    return _sc_gather(user_batch.astype(jnp.int32),
                      item_batch.astype(jnp.int32), rs_u, rs_i)
